# baseline (device time: 15107 ns/iter reference)
import jax
import jax.numpy as jnp
from jax import lax
from jax.experimental import pallas as pl
from jax.experimental.pallas import tpu as pltpu

N_DEV = 4


def kernel(ids, E):
    v_per, d = E.shape

    my_pos = lax.axis_index("i")
    local = ids - my_pos * v_per
    rows = jnp.take(E, jnp.clip(local, 0, v_per - 1), axis=0)
    return _direct_all_reduce(rows, local[:, None], v_per)


N_HALF = 4


def _direct_all_reduce(x, local2d, v_per):
    t, d = x.shape
    chunk = t // N_DEV
    dh = d // N_HALF

    def body(x_ref, loc_ref, out_ref, xb, rs_buf, ag_buf, red_bf, own_f32,
             agf, rs_send_sems, rs_recv_sems, ag_send_sems, ag_recv_sems,
             out_sems):
        my = lax.axis_index("i")

        barrier_sem = pltpu.get_barrier_semaphore()
        for k in range(1, N_DEV):
            peer = lax.rem(my + k, N_DEV)
            pl.semaphore_signal(
                barrier_sem, inc=1,
                device_id=(peer,), device_id_type=pl.DeviceIdType.MESH,
            )
        pl.semaphore_wait(barrier_sem, N_DEV - 1)

        loc = loc_ref[...]
        maskf = ((loc >= 0) & (loc < v_per)).astype(jnp.float32)

        rs = [[None] * (N_DEV - 1) for _ in range(N_HALF)]
        for h in range(N_HALF):
            cs = pl.ds(h * dh, dh)
            xb[:, cs] = (x_ref[:, cs] * maskf).astype(jnp.bfloat16)
            for k in range(1, N_DEV):
                peer = lax.rem(my + k, N_DEV)
                r = pltpu.make_async_remote_copy(
                    src_ref=xb.at[pl.ds(peer * chunk, chunk), cs],
                    dst_ref=rs_buf.at[k - 1, :, cs],
                    send_sem=rs_send_sems.at[h, k - 1],
                    recv_sem=rs_recv_sems.at[h, k - 1],
                    device_id=(peer,),
                    device_id_type=pl.DeviceIdType.MESH,
                )
                r.start()
                rs[h][k - 1] = r
        ag = [[None] * (N_DEV - 1) for _ in range(N_HALF)]
        for h in range(N_HALF):
            cs = pl.ds(h * dh, dh)
            for r in rs[h]:
                r.wait_recv()
            acc = (xb[pl.ds(my * chunk, chunk), cs].astype(jnp.float32)
                   + rs_buf[0, :, cs].astype(jnp.float32)
                   + rs_buf[1, :, cs].astype(jnp.float32)
                   + rs_buf[2, :, cs].astype(jnp.float32))
            own_f32[:, cs] = acc
            red_bf[:, cs] = acc.astype(jnp.bfloat16)
            for k in range(1, N_DEV):
                peer = lax.rem(my + k, N_DEV)
                r = pltpu.make_async_remote_copy(
                    src_ref=red_bf.at[:, cs],
                    dst_ref=ag_buf.at[k - 1, :, cs],
                    send_sem=ag_send_sems.at[h, k - 1],
                    recv_sem=ag_recv_sems.at[h, k - 1],
                    device_id=(peer,),
                    device_id_type=pl.DeviceIdType.MESH,
                )
                r.start()
                ag[h][k - 1] = r
        own_copy = pltpu.make_async_copy(
            own_f32, out_ref.at[pl.ds(my * chunk, chunk), :], out_sems.at[0])
        own_copy.start()

        out_copies = [own_copy]
        for h in range(N_HALF):
            for k in range(1, N_DEV):
                ag[h][k - 1].wait_recv()
        for k in range(1, N_DEV):
            src = lax.rem(my - k + 2 * N_DEV, N_DEV)
            agf[k - 1] = ag_buf[k - 1].astype(jnp.float32)
            c = pltpu.make_async_copy(
                agf.at[k - 1], out_ref.at[pl.ds(src * chunk, chunk), :],
                out_sems.at[k])
            c.start()
            out_copies.append(c)
        for h in range(N_HALF):
            for r in rs[h]:
                r.wait_send()
            for r in ag[h]:
                r.wait_send()
        for c in out_copies:
            c.wait()

    return pl.pallas_call(
        body,
        out_shape=jax.ShapeDtypeStruct((t, d), jnp.float32),
        in_specs=[pl.BlockSpec(memory_space=pltpu.VMEM),
                  pl.BlockSpec(memory_space=pltpu.VMEM)],
        out_specs=pl.BlockSpec(memory_space=pltpu.MemorySpace.HBM),
        scratch_shapes=[
            pltpu.VMEM((t, d), jnp.bfloat16),
            pltpu.VMEM((N_DEV - 1, chunk, d), jnp.bfloat16),
            pltpu.VMEM((N_DEV - 1, chunk, d), jnp.bfloat16),
            pltpu.VMEM((chunk, d), jnp.bfloat16),
            pltpu.VMEM((chunk, d), jnp.float32),
            pltpu.VMEM((N_DEV - 1, chunk, d), jnp.float32),
            pltpu.SemaphoreType.DMA((N_HALF, N_DEV - 1)),
            pltpu.SemaphoreType.DMA((N_HALF, N_DEV - 1)),
            pltpu.SemaphoreType.DMA((N_HALF, N_DEV - 1)),
            pltpu.SemaphoreType.DMA((N_HALF, N_DEV - 1)),
            pltpu.SemaphoreType.DMA((N_DEV,)),
        ],
        compiler_params=pltpu.CompilerParams(collective_id=0),
    )(x, local2d)


# device time: 14803 ns/iter; 1.0205x vs baseline; 1.0205x over previous
import jax
import jax.numpy as jnp
from jax import lax
from jax.experimental import pallas as pl
from jax.experimental.pallas import tpu as pltpu

N_DEV = 4


def kernel(ids, E):
    v_per, d = E.shape

    my_pos = lax.axis_index("i")
    local = ids - my_pos * v_per
    in_range = (local >= 0) & (local < v_per)
    safe = jnp.where(in_range, local, 0)
    partial = jnp.take(E, safe, axis=0) * in_range[:, None].astype(E.dtype)
    return _direct_all_reduce(partial)


N_HALF = 4


def _direct_all_reduce(x):
    t, d = x.shape
    chunk = t // N_DEV
    dh = d // N_HALF

    def body(x_ref, out_ref, xb, rs_buf, ag_buf, red_bf,
             rs_send_sems, rs_recv_sems, ag_send_sems, ag_recv_sems):
        my = lax.axis_index("i")

        barrier_sem = pltpu.get_barrier_semaphore()
        for k in range(1, N_DEV):
            peer = lax.rem(my + k, N_DEV)
            pl.semaphore_signal(
                barrier_sem, inc=1,
                device_id=(peer,), device_id_type=pl.DeviceIdType.MESH,
            )
        pl.semaphore_wait(barrier_sem, N_DEV - 1)

        rs = [[None] * (N_DEV - 1) for _ in range(N_HALF)]
        for h in range(N_HALF):
            cs = pl.ds(h * dh, dh)
            xb[:, cs] = x_ref[:, cs].astype(jnp.bfloat16)
            for k in range(1, N_DEV):
                peer = lax.rem(my + k, N_DEV)
                r = pltpu.make_async_remote_copy(
                    src_ref=xb.at[pl.ds(peer * chunk, chunk), cs],
                    dst_ref=rs_buf.at[k - 1, :, cs],
                    send_sem=rs_send_sems.at[h, k - 1],
                    recv_sem=rs_recv_sems.at[h, k - 1],
                    device_id=(peer,),
                    device_id_type=pl.DeviceIdType.MESH,
                )
                r.start()
                rs[h][k - 1] = r
        ag = [[None] * (N_DEV - 1) for _ in range(N_HALF)]
        for h in range(N_HALF):
            cs = pl.ds(h * dh, dh)
            for r in rs[h]:
                r.wait_recv()
            acc = (x_ref[pl.ds(my * chunk, chunk), cs]
                   + rs_buf[0, :, cs].astype(jnp.float32)
                   + rs_buf[1, :, cs].astype(jnp.float32)
                   + rs_buf[2, :, cs].astype(jnp.float32))
            out_ref[pl.ds(my * chunk, chunk), cs] = acc
            red_bf[:, cs] = acc.astype(jnp.bfloat16)
            for k in range(1, N_DEV):
                peer = lax.rem(my + k, N_DEV)
                r = pltpu.make_async_remote_copy(
                    src_ref=red_bf.at[:, cs],
                    dst_ref=ag_buf.at[k - 1, :, cs],
                    send_sem=ag_send_sems.at[h, k - 1],
                    recv_sem=ag_recv_sems.at[h, k - 1],
                    device_id=(peer,),
                    device_id_type=pl.DeviceIdType.MESH,
                )
                r.start()
                ag[h][k - 1] = r
        for h in range(N_HALF):
            cs = pl.ds(h * dh, dh)
            for k in range(1, N_DEV):
                src = lax.rem(my - k + 2 * N_DEV, N_DEV)
                ag[h][k - 1].wait_recv()
                out_ref[pl.ds(src * chunk, chunk), cs] = (
                    ag_buf[k - 1, :, cs].astype(jnp.float32))
        for h in range(N_HALF):
            for r in rs[h]:
                r.wait_send()
            for r in ag[h]:
                r.wait_send()

    return pl.pallas_call(
        body,
        out_shape=jax.ShapeDtypeStruct((t, d), jnp.float32),
        in_specs=[pl.BlockSpec(memory_space=pltpu.VMEM)],
        out_specs=pl.BlockSpec(memory_space=pltpu.VMEM),
        scratch_shapes=[
            pltpu.VMEM((t, d), jnp.bfloat16),
            pltpu.VMEM((N_DEV - 1, chunk, d), jnp.bfloat16),
            pltpu.VMEM((N_DEV - 1, chunk, d), jnp.bfloat16),
            pltpu.VMEM((chunk, d), jnp.bfloat16),
            pltpu.SemaphoreType.DMA((N_HALF, N_DEV - 1)),
            pltpu.SemaphoreType.DMA((N_HALF, N_DEV - 1)),
            pltpu.SemaphoreType.DMA((N_HALF, N_DEV - 1)),
            pltpu.SemaphoreType.DMA((N_HALF, N_DEV - 1)),
        ],
        compiler_params=pltpu.CompilerParams(collective_id=0),
    )(x)


# device time: 14790 ns/iter; 1.0214x vs baseline; 1.0009x over previous
import jax
import jax.numpy as jnp
from jax import lax
from jax.experimental import pallas as pl
from jax.experimental.pallas import tpu as pltpu

N_DEV = 4


def kernel(ids, E):
    v_per, d = E.shape

    my_pos = lax.axis_index("i")
    local = ids - my_pos * v_per
    in_range = (local >= 0) & (local < v_per)
    safe = jnp.where(in_range, local, 0)
    partial = jnp.take(E, safe, axis=0) * in_range[:, None].astype(E.dtype)
    return _direct_all_reduce(partial)


N_HALF = 4


def _direct_all_reduce(x):
    t, d = x.shape
    chunk = t // N_DEV
    dh = d // N_HALF

    def body(x_ref, out_ref, xv, xb, rs_buf, ag_buf, red_bf,
             rs_send_sems, rs_recv_sems, ag_send_sems, ag_recv_sems,
             in_sem):
        my = lax.axis_index("i")

        barrier_sem = pltpu.get_barrier_semaphore()
        for k in range(1, N_DEV):
            peer = lax.rem(my + k, N_DEV)
            pl.semaphore_signal(
                barrier_sem, inc=1,
                device_id=(peer,), device_id_type=pl.DeviceIdType.MESH,
            )
        in_copy = pltpu.make_async_copy(x_ref, xv, in_sem)
        in_copy.start()
        in_copy.wait()
        pl.semaphore_wait(barrier_sem, N_DEV - 1)

        rs = [[None] * (N_DEV - 1) for _ in range(N_HALF)]
        for h in range(N_HALF):
            cs = pl.ds(h * dh, dh)
            xb[:, cs] = xv[:, cs].astype(jnp.bfloat16)
            for k in range(1, N_DEV):
                peer = lax.rem(my + k, N_DEV)
                r = pltpu.make_async_remote_copy(
                    src_ref=xb.at[pl.ds(peer * chunk, chunk), cs],
                    dst_ref=rs_buf.at[k - 1, :, cs],
                    send_sem=rs_send_sems.at[h, k - 1],
                    recv_sem=rs_recv_sems.at[h, k - 1],
                    device_id=(peer,),
                    device_id_type=pl.DeviceIdType.MESH,
                )
                r.start()
                rs[h][k - 1] = r
        ag = [[None] * (N_DEV - 1) for _ in range(N_HALF)]
        for h in range(N_HALF):
            cs = pl.ds(h * dh, dh)
            for r in rs[h]:
                r.wait_recv()
            acc = (xv[pl.ds(my * chunk, chunk), cs]
                   + rs_buf[0, :, cs].astype(jnp.float32)
                   + rs_buf[1, :, cs].astype(jnp.float32)
                   + rs_buf[2, :, cs].astype(jnp.float32))
            out_ref[pl.ds(my * chunk, chunk), cs] = acc
            red_bf[:, cs] = acc.astype(jnp.bfloat16)
            for k in range(1, N_DEV):
                peer = lax.rem(my + k, N_DEV)
                r = pltpu.make_async_remote_copy(
                    src_ref=red_bf.at[:, cs],
                    dst_ref=ag_buf.at[k - 1, :, cs],
                    send_sem=ag_send_sems.at[h, k - 1],
                    recv_sem=ag_recv_sems.at[h, k - 1],
                    device_id=(peer,),
                    device_id_type=pl.DeviceIdType.MESH,
                )
                r.start()
                ag[h][k - 1] = r
        for h in range(N_HALF):
            cs = pl.ds(h * dh, dh)
            for k in range(1, N_DEV):
                src = lax.rem(my - k + 2 * N_DEV, N_DEV)
                ag[h][k - 1].wait_recv()
                out_ref[pl.ds(src * chunk, chunk), cs] = (
                    ag_buf[k - 1, :, cs].astype(jnp.float32))
        for h in range(N_HALF):
            for r in rs[h]:
                r.wait_send()
            for r in ag[h]:
                r.wait_send()

    return pl.pallas_call(
        body,
        out_shape=jax.ShapeDtypeStruct((t, d), jnp.float32),
        in_specs=[pl.BlockSpec(memory_space=pl.ANY)],
        out_specs=pl.BlockSpec(memory_space=pltpu.VMEM),
        scratch_shapes=[
            pltpu.VMEM((t, d), jnp.float32),
            pltpu.VMEM((t, d), jnp.bfloat16),
            pltpu.VMEM((N_DEV - 1, chunk, d), jnp.bfloat16),
            pltpu.VMEM((N_DEV - 1, chunk, d), jnp.bfloat16),
            pltpu.VMEM((chunk, d), jnp.bfloat16),
            pltpu.SemaphoreType.DMA((N_HALF, N_DEV - 1)),
            pltpu.SemaphoreType.DMA((N_HALF, N_DEV - 1)),
            pltpu.SemaphoreType.DMA((N_HALF, N_DEV - 1)),
            pltpu.SemaphoreType.DMA((N_HALF, N_DEV - 1)),
            pltpu.SemaphoreType.DMA,
        ],
        compiler_params=pltpu.CompilerParams(collective_id=0),
    )(x)
